# Initial kernel scaffold; baseline (speedup 1.0000x reference)
import jax
import jax.numpy as jnp
from jax.experimental import pallas as pl


def kernel(x_user, x_item, edge_index_u2i, edge_index_i2u, W0_u2i, b0_u2i, W0_i2u, b0_i2u, W1_u2i, b1_u2i, W1_i2u, b1_i2u):
    raise NotImplementedError("write your pallas kernel here")


# speedup vs baseline: 4.0344x; 4.0344x over previous
import functools

import jax
import jax.numpy as jnp
from jax import lax
from jax.experimental import pallas as pl
from jax.experimental.pallas import tpu as pltpu
from jax.experimental.pallas import tpu_sc as plsc

N = 10000
E = 320000
D = 128
NC = 2
NS = 16
B = 128
NSUB = 2
SB = B // NSUB
SPT = 160
IBK = 8
NBLK = SPT // IBK
EPT = SPT * B
E_PAD = EPT * NS
NPAD = N + 16
RPT = 624
TAIL_O = N - RPT * NS
TAIL_Z = NPAD - RPT * NS


def _tc_layer(x2n, Wa, ba, Wb, bb, swap):
  def body(x_ref, wa_ref, ba_ref, wb_ref, bb_ref, o_ref):
    if swap:
      xa = x_ref[N:, :]
      xb = x_ref[:N, :]
    else:
      xa = x_ref[:N, :]
      xb = x_ref[N:, :]
    ha = jnp.dot(xa, wa_ref[...], preferred_element_type=jnp.float32,
                 precision=lax.Precision.HIGHEST)
    o_ref[:N, :] = jnp.maximum(ha + ba_ref[...], 0.0)
    hb = jnp.dot(xb, wb_ref[...], preferred_element_type=jnp.float32,
                 precision=lax.Precision.HIGHEST)
    o_ref[N:, :] = jnp.maximum(hb + bb_ref[...], 0.0)

  return pl.pallas_call(
      body,
      out_shape=jax.ShapeDtypeStruct((2 * N, D), jnp.float32),
  )(x2n, Wa, ba.reshape(1, D), Wb, bb.reshape(1, D))


def _sc_agg(h2n, idx_all, zrows):
  mesh = plsc.VectorSubcoreMesh(core_axis_name="c", subcore_axis_name="s")

  @functools.partial(
      pl.kernel,
      mesh=mesh,
      out_type=jax.ShapeDtypeStruct((2 * N, D), jnp.float32),
      scratch_types=[
          pltpu.VMEM((2 * IBK, B), jnp.int32),
          pltpu.VMEM((B, D), jnp.float32),
          pltpu.VMEM((B, D), jnp.float32),
          pltpu.VMEM_SHARED((NPAD, D), jnp.float32),
          pltpu.SemaphoreType.DMA,
          pltpu.SemaphoreType.DMA,
          pltpu.SemaphoreType.DMA,
          pltpu.SemaphoreType.DMA,
      ],
  )
  def k(h_hbm, idx_hbm, z_hbm, out_hbm, idx_v,
        rows0, rows1, acc, gsem0, gsem1, ssem0, ssem1):
    c = lax.axis_index("c")
    s = lax.axis_index("s")

    pltpu.sync_copy(z_hbm, acc.at[pl.ds(s * RPT, RPT)])

    @pl.when(s == NS - 1)
    def _():
      pltpu.sync_copy(z_hbm.at[pl.ds(0, TAIL_Z)], acc.at[pl.ds(NS * RPT, TAIL_Z)])

    plsc.subcore_barrier()

    def g_start(j, buf, sem):
      for m in range(NSUB):
        pltpu.async_copy(h_hbm.at[idx_v.at[j, pl.ds(m * SB, SB)]],
                         buf.at[pl.ds(m * SB, SB)], sem)

    def g_wait(j, buf, sem):
      for m in range(NSUB):
        pltpu.make_async_copy(h_hbm.at[idx_v.at[j, pl.ds(m * SB, SB)]],
                              buf.at[pl.ds(m * SB, SB)], sem).wait()

    def s_start(j, buf, sem):
      pltpu.async_copy(buf, acc.at[idx_v.at[IBK + j]], sem, add=True)

    def s_wait(j, buf, sem):
      pltpu.make_async_copy(buf, acc.at[idx_v.at[IBK + j]], sem).wait()

    @pl.loop(0, NBLK)
    def _(i):
      pltpu.sync_copy(idx_hbm.at[c, s, i], idx_v)

      g_start(0, rows0, gsem0)
      for p in range(IBK // 2 - 1):
        j = 2 * p
        g_wait(j, rows0, gsem0)
        s_start(j, rows0, ssem0)
        if j >= 1:
          s_wait(j - 1, rows1, ssem1)
        g_start(j + 1, rows1, gsem1)
        g_wait(j + 1, rows1, gsem1)
        s_start(j + 1, rows1, ssem1)
        s_wait(j, rows0, ssem0)
        g_start(j + 2, rows0, gsem0)
      g_wait(IBK - 2, rows0, gsem0)
      s_start(IBK - 2, rows0, ssem0)
      s_wait(IBK - 3, rows1, ssem1)
      g_start(IBK - 1, rows1, gsem1)
      g_wait(IBK - 1, rows1, gsem1)
      s_start(IBK - 1, rows1, ssem1)
      s_wait(IBK - 2, rows0, ssem0)
      s_wait(IBK - 1, rows1, ssem1)

    plsc.subcore_barrier()
    pltpu.sync_copy(acc.at[pl.ds(s * RPT, RPT)],
                    out_hbm.at[pl.ds(c * N + s * RPT, RPT)])

    @pl.when(s == NS - 1)
    def _():
      pltpu.sync_copy(acc.at[pl.ds(NS * RPT, TAIL_O)],
                      out_hbm.at[pl.ds(c * N + NS * RPT, TAIL_O)])

  return k(h2n, idx_all, zrows)


def _prep_edges(ei_u2i, ei_i2u):
  pad = E_PAD - E
  pad_src = jnp.zeros((pad,), jnp.int32)
  pad_dst = jnp.full((pad,), N, jnp.int32)
  srcs = []
  dsts = []
  for rel, ei in enumerate((ei_u2i, ei_i2u)):
    src = ei[0].astype(jnp.int32) + rel * N
    dst = ei[1].astype(jnp.int32)
    srcs.append(jnp.concatenate([src, pad_src + rel * N]))
    dsts.append(jnp.concatenate([dst, pad_dst]))
  src_idx = jnp.stack(srcs).reshape(NC, NS, NBLK, IBK, B)
  dst_idx = jnp.stack(dsts).reshape(NC, NS, NBLK, IBK, B)
  return jnp.concatenate([src_idx, dst_idx], axis=3)


def kernel(x_user, x_item, edge_index_u2i, edge_index_i2u,
           W0_u2i, b0_u2i, W0_i2u, b0_i2u,
           W1_u2i, b1_u2i, W1_i2u, b1_i2u):
  idx_all = _prep_edges(edge_index_u2i, edge_index_i2u)
  zrows = jnp.zeros((RPT, D), jnp.float32)

  x2n = jnp.concatenate([x_user, x_item], axis=0)
  h0 = _tc_layer(x2n, W0_u2i, b0_u2i, W0_i2u, b0_i2u, swap=False)
  agg0 = _sc_agg(h0, idx_all, zrows)
  h1 = _tc_layer(agg0, W1_u2i, b1_u2i, W1_i2u, b1_i2u, swap=True)
  agg1 = _sc_agg(h1, idx_all, zrows)
  return agg1[N:], agg1[:N]


# speedup vs baseline: 6.1764x; 1.5309x over previous
import functools

import jax
import jax.numpy as jnp
from jax import lax
from jax.experimental import pallas as pl
from jax.experimental.pallas import tpu as pltpu
from jax.experimental.pallas import tpu_sc as plsc

N = 10000
E = 320000
D = 128
NC = 2
NS = 16
B = 128
SPT = 158
IBK = 16
NBLK = SPT // IBK
TBK = SPT - NBLK * IBK
EPT = SPT * B
EPT_REAL = E // NS
E_PAD = EPT * NS
NPAD = N + 16
RPT = 624
TAIL_O = N - RPT * NS
TAIL_Z = NPAD - RPT * NS


def _tc_layer(x2n, Wa, ba, Wb, bb, swap):
  def body(x_ref, wa_ref, ba_ref, wb_ref, bb_ref, o_ref):
    if swap:
      xa = x_ref[N:, :]
      xb = x_ref[:N, :]
    else:
      xa = x_ref[:N, :]
      xb = x_ref[N:, :]
    ha = jnp.dot(xa, wa_ref[...], preferred_element_type=jnp.float32,
                 precision=lax.Precision.HIGHEST)
    o_ref[:N, :] = jnp.maximum(ha + ba_ref[...], 0.0)
    hb = jnp.dot(xb, wb_ref[...], preferred_element_type=jnp.float32,
                 precision=lax.Precision.HIGHEST)
    o_ref[N:, :] = jnp.maximum(hb + bb_ref[...], 0.0)

  return pl.pallas_call(
      body,
      out_shape=jax.ShapeDtypeStruct((2 * N, D), jnp.float32),
  )(x2n, Wa, ba.reshape(1, D), Wb, bb.reshape(1, D))


def _sc_agg(h2n, idx_all, zrows):
  mesh = plsc.VectorSubcoreMesh(core_axis_name="c", subcore_axis_name="s")

  @functools.partial(
      pl.kernel,
      mesh=mesh,
      out_type=jax.ShapeDtypeStruct((2 * N, D), jnp.float32),
      scratch_types=[
          pltpu.VMEM((2 * IBK, B), jnp.int32),
          pltpu.VMEM((B, D), jnp.float32),
          pltpu.VMEM((B, D), jnp.float32),
          pltpu.VMEM_SHARED((NPAD, D), jnp.float32),
          pltpu.SemaphoreType.DMA,
          pltpu.SemaphoreType.DMA,
          pltpu.SemaphoreType.DMA,
          pltpu.SemaphoreType.DMA,
      ],
  )
  def k(h_hbm, idx_hbm, z_hbm, out_hbm, idx_v,
        rows0, rows1, acc, gsem0, gsem1, ssem0, ssem1):
    c = lax.axis_index("c")
    s = lax.axis_index("s")

    pltpu.sync_copy(z_hbm, acc.at[pl.ds(s * RPT, RPT)])

    @pl.when(s == NS - 1)
    def _():
      pltpu.sync_copy(z_hbm.at[pl.ds(0, TAIL_Z)], acc.at[pl.ds(NS * RPT, TAIL_Z)])

    plsc.subcore_barrier()

    def g_start(j, buf, sem):
      pltpu.async_copy(h_hbm.at[idx_v.at[j]], buf, sem)

    def g_wait(j, buf, sem):
      pltpu.make_async_copy(h_hbm.at[idx_v.at[j]], buf, sem).wait()

    def s_start(j, buf, sem):
      pltpu.async_copy(buf, acc.at[idx_v.at[IBK + j]], sem, add=True)

    def s_wait(j, buf, sem):
      pltpu.make_async_copy(buf, acc.at[idx_v.at[IBK + j]], sem).wait()

    def process_block(nstreams):
      g_start(0, rows0, gsem0)
      for p in range(nstreams // 2 - 1):
        j = 2 * p
        g_wait(j, rows0, gsem0)
        s_start(j, rows0, ssem0)
        if j >= 1:
          s_wait(j - 1, rows1, ssem1)
        g_start(j + 1, rows1, gsem1)
        g_wait(j + 1, rows1, gsem1)
        s_start(j + 1, rows1, ssem1)
        s_wait(j, rows0, ssem0)
        g_start(j + 2, rows0, gsem0)
      g_wait(nstreams - 2, rows0, gsem0)
      s_start(nstreams - 2, rows0, ssem0)
      s_wait(nstreams - 3, rows1, ssem1)
      g_start(nstreams - 1, rows1, gsem1)
      g_wait(nstreams - 1, rows1, gsem1)
      s_start(nstreams - 1, rows1, ssem1)
      s_wait(nstreams - 2, rows0, ssem0)
      s_wait(nstreams - 1, rows1, ssem1)

    @pl.loop(0, NBLK)
    def _(i):
      pltpu.sync_copy(idx_hbm.at[c, s, i], idx_v)
      process_block(IBK)

    pltpu.sync_copy(idx_hbm.at[c, s, NBLK], idx_v)
    process_block(TBK)

    plsc.subcore_barrier()
    pltpu.sync_copy(acc.at[pl.ds(s * RPT, RPT)],
                    out_hbm.at[pl.ds(c * N + s * RPT, RPT)])

    @pl.when(s == NS - 1)
    def _():
      pltpu.sync_copy(acc.at[pl.ds(NS * RPT, TAIL_O)],
                      out_hbm.at[pl.ds(c * N + NS * RPT, TAIL_O)])

  return k(h2n, idx_all, zrows)


def _prep_edges(ei_u2i, ei_i2u):
  nblk_t = NBLK + 1
  pad_s = EPT - EPT_REAL
  pad_b = (nblk_t * IBK - SPT) * B

  def layout(flat, fill):
    per_tile = flat.reshape(NS, EPT_REAL)
    per_tile = jnp.concatenate(
        [per_tile, jnp.full((NS, pad_s + pad_b), fill, jnp.int32)], axis=1)
    return per_tile.reshape(NS, nblk_t, IBK, B)

  blocks = []
  for rel, ei in enumerate((ei_u2i, ei_i2u)):
    src = layout(ei[0].astype(jnp.int32) + rel * N, rel * N)
    dst = layout(ei[1].astype(jnp.int32), N)
    blocks.append(jnp.concatenate([src, dst], axis=2))
  return jnp.stack(blocks)


def kernel(x_user, x_item, edge_index_u2i, edge_index_i2u,
           W0_u2i, b0_u2i, W0_i2u, b0_i2u,
           W1_u2i, b1_u2i, W1_i2u, b1_i2u):
  idx_all = _prep_edges(edge_index_u2i, edge_index_i2u)
  zrows = jnp.zeros((RPT, D), jnp.float32)

  x2n = jnp.concatenate([x_user, x_item], axis=0)
  h0 = _tc_layer(x2n, W0_u2i, b0_u2i, W0_i2u, b0_i2u, swap=False)
  agg0 = _sc_agg(h0, idx_all, zrows)
  h1 = _tc_layer(agg0, W1_u2i, b1_u2i, W1_i2u, b1_i2u, swap=True)
  agg1 = _sc_agg(h1, idx_all, zrows)
  return agg1[N:], agg1[:N]


# speedup vs baseline: 6.2332x; 1.0092x over previous
import functools

import jax
import jax.numpy as jnp
from jax import lax
from jax.experimental import pallas as pl
from jax.experimental.pallas import tpu as pltpu
from jax.experimental.pallas import tpu_sc as plsc

N = 10000
E = 320000
D = 128
NC = 2
NS = 16
B = 128
SPT = 158
IBK = 16
NBLK = SPT // IBK
TBK = SPT - NBLK * IBK
EPT = SPT * B
EPT_REAL = E // NS
E_PAD = EPT * NS
NPAD = N + 16
RPT = 624
TAIL_O = N - RPT * NS
TAIL_Z = NPAD - RPT * NS


def _tc_layer(x2n, Wa, ba, Wb, bb, swap):
  def body(x_ref, wa_ref, ba_ref, wb_ref, bb_ref, o_ref):
    if swap:
      xa = x_ref[N:, :]
      xb = x_ref[:N, :]
    else:
      xa = x_ref[:N, :]
      xb = x_ref[N:, :]
    ha = jnp.dot(xa, wa_ref[...], preferred_element_type=jnp.float32,
                 precision=lax.Precision.HIGHEST)
    o_ref[:N, :] = jnp.maximum(ha + ba_ref[...], 0.0)
    hb = jnp.dot(xb, wb_ref[...], preferred_element_type=jnp.float32,
                 precision=lax.Precision.HIGHEST)
    o_ref[N:, :] = jnp.maximum(hb + bb_ref[...], 0.0)

  return pl.pallas_call(
      body,
      out_shape=jax.ShapeDtypeStruct((2 * N, D), jnp.float32),
  )(x2n, Wa, ba.reshape(1, D), Wb, bb.reshape(1, D))


def _sc_agg(h2n, idx_all, zrows):
  mesh = plsc.VectorSubcoreMesh(core_axis_name="c", subcore_axis_name="s")

  @functools.partial(
      pl.kernel,
      mesh=mesh,
      out_type=jax.ShapeDtypeStruct((2 * N, D), jnp.float32),
      scratch_types=[
          pltpu.VMEM((2 * IBK, B), jnp.int32),
          pltpu.VMEM((B, D), jnp.float32),
          pltpu.VMEM((B, D), jnp.float32),
          pltpu.VMEM_SHARED((NPAD, D), jnp.float32),
          pltpu.SemaphoreType.DMA,
          pltpu.SemaphoreType.DMA,
          pltpu.SemaphoreType.DMA,
          pltpu.SemaphoreType.DMA,
      ],
  )
  def k(h_hbm, idx_hbm, z_hbm, out_hbm, idx_v,
        rows0, rows1, acc, gsem0, gsem1, ssem0, ssem1):
    c = lax.axis_index("c")
    s = lax.axis_index("s")

    pltpu.sync_copy(z_hbm, acc.at[pl.ds(s * RPT, RPT)])

    @pl.when(s == NS - 1)
    def _():
      pltpu.sync_copy(z_hbm.at[pl.ds(0, TAIL_Z)], acc.at[pl.ds(NS * RPT, TAIL_Z)])

    plsc.subcore_barrier()

    def g_start(j, buf, sem):
      pltpu.async_copy(h_hbm.at[idx_v.at[j]], buf, sem)

    def g_wait(j, buf, sem):
      pltpu.make_async_copy(h_hbm.at[idx_v.at[j]], buf, sem).wait()

    def s_start(j, buf, sem):
      pltpu.async_copy(buf, acc.at[idx_v.at[IBK + j]], sem, add=True)

    def s_wait(j, buf, sem):
      pltpu.make_async_copy(buf, acc.at[idx_v.at[IBK + j]], sem).wait()

    def process_block(nstreams):
      g_start(0, rows0, gsem0)
      for p in range(nstreams // 2 - 1):
        j = 2 * p
        g_wait(j, rows0, gsem0)
        s_start(j, rows0, ssem0)
        if j >= 1:
          s_wait(j - 1, rows1, ssem1)
        g_start(j + 1, rows1, gsem1)
        g_wait(j + 1, rows1, gsem1)
        s_start(j + 1, rows1, ssem1)
        s_wait(j, rows0, ssem0)
        g_start(j + 2, rows0, gsem0)
      g_wait(nstreams - 2, rows0, gsem0)
      s_start(nstreams - 2, rows0, ssem0)
      s_wait(nstreams - 3, rows1, ssem1)
      g_start(nstreams - 1, rows1, gsem1)
      g_wait(nstreams - 1, rows1, gsem1)
      s_start(nstreams - 1, rows1, ssem1)
      s_wait(nstreams - 2, rows0, ssem0)
      s_wait(nstreams - 1, rows1, ssem1)

    @pl.loop(0, NBLK)
    def _(i):
      pltpu.sync_copy(idx_hbm.at[c, s, i], idx_v)
      process_block(IBK)

    pltpu.sync_copy(idx_hbm.at[c, s, NBLK], idx_v)
    process_block(TBK)

    plsc.subcore_barrier()
    pltpu.sync_copy(acc.at[pl.ds(s * RPT, RPT)],
                    out_hbm.at[pl.ds(c * N + s * RPT, RPT)])

    @pl.when(s == NS - 1)
    def _():
      pltpu.sync_copy(acc.at[pl.ds(NS * RPT, TAIL_O)],
                      out_hbm.at[pl.ds(c * N + NS * RPT, TAIL_O)])

  return k(h2n, idx_all, zrows)


def _prep_edges(ei_u2i, ei_i2u):
  nblk_t = NBLK + 1
  pad_s = EPT - EPT_REAL
  pad_b = (nblk_t * IBK - SPT) * B

  def layout(flat, fill):
    per_tile = flat.reshape(NS, EPT_REAL)
    per_tile = jnp.concatenate(
        [per_tile, jnp.broadcast_to(fill, (NS, pad_s + pad_b)).astype(jnp.int32)],
        axis=1)
    return per_tile.reshape(NS, nblk_t, IBK, B)

  pad_dst = N + jnp.arange(pad_s + pad_b, dtype=jnp.int32) % (NPAD - N)
  blocks = []
  for rel, ei in enumerate((ei_u2i, ei_i2u)):
    src = layout(ei[0].astype(jnp.int32) + rel * N, rel * N)
    dst = layout(ei[1].astype(jnp.int32), pad_dst)
    blocks.append(jnp.concatenate([src, dst], axis=2))
  return jnp.stack(blocks)


def kernel(x_user, x_item, edge_index_u2i, edge_index_i2u,
           W0_u2i, b0_u2i, W0_i2u, b0_i2u,
           W1_u2i, b1_u2i, W1_i2u, b1_i2u):
  idx_all = _prep_edges(edge_index_u2i, edge_index_i2u)
  zrows = jnp.zeros((RPT, D), jnp.float32)

  x2n = jnp.concatenate([x_user, x_item], axis=0)
  h0 = _tc_layer(x2n, W0_u2i, b0_u2i, W0_i2u, b0_i2u, swap=False)
  agg0 = _sc_agg(h0, idx_all, zrows)
  h1 = _tc_layer(agg0, W1_u2i, b1_u2i, W1_i2u, b1_i2u, swap=True)
  agg1 = _sc_agg(h1, idx_all, zrows)
  return agg1[N:], agg1[:N]


# speedup vs baseline: 6.3094x; 1.0122x over previous
import functools

import jax
import jax.numpy as jnp
from jax import lax
from jax.experimental import pallas as pl
from jax.experimental.pallas import tpu as pltpu
from jax.experimental.pallas import tpu_sc as plsc

N = 10000
E = 320000
D = 128
NC = 2
NS = 16
B = 128
SPT = 158
IBK = 16
NBLK = SPT // IBK
TBK = SPT - NBLK * IBK
EPT = SPT * B
EPT_REAL = E // NS
E_PAD = EPT * NS
NPAD = N + 16
RPT = 624
TAIL_O = N - RPT * NS
TAIL_Z = NPAD - RPT * NS


def _tc_layer(x2n, Wa, ba, Wb, bb, swap):
  def body(x_ref, wa_ref, ba_ref, wb_ref, bb_ref, o_ref):
    if swap:
      xa = x_ref[N:, :]
      xb = x_ref[:N, :]
    else:
      xa = x_ref[:N, :]
      xb = x_ref[N:, :]
    ha = jnp.dot(xa, wa_ref[...], preferred_element_type=jnp.float32)
    o_ref[:N, :] = jnp.maximum(ha + ba_ref[...], 0.0)
    hb = jnp.dot(xb, wb_ref[...], preferred_element_type=jnp.float32)
    o_ref[N:, :] = jnp.maximum(hb + bb_ref[...], 0.0)

  return pl.pallas_call(
      body,
      out_shape=jax.ShapeDtypeStruct((2 * N, D), jnp.float32),
  )(x2n, Wa, ba.reshape(1, D), Wb, bb.reshape(1, D))


def _sc_agg(h2n, idx_all, zrows):
  mesh = plsc.VectorSubcoreMesh(core_axis_name="c", subcore_axis_name="s")

  @functools.partial(
      pl.kernel,
      mesh=mesh,
      out_type=jax.ShapeDtypeStruct((2 * N, D), jnp.float32),
      scratch_types=[
          pltpu.VMEM((2 * IBK, B), jnp.int32),
          pltpu.VMEM((B, D), jnp.float32),
          pltpu.VMEM((B, D), jnp.float32),
          pltpu.VMEM_SHARED((NPAD, D), jnp.float32),
          pltpu.SemaphoreType.DMA,
          pltpu.SemaphoreType.DMA,
          pltpu.SemaphoreType.DMA,
          pltpu.SemaphoreType.DMA,
      ],
  )
  def k(h_hbm, idx_hbm, z_hbm, out_hbm, idx_v,
        rows0, rows1, acc, gsem0, gsem1, ssem0, ssem1):
    c = lax.axis_index("c")
    s = lax.axis_index("s")

    pltpu.sync_copy(z_hbm, acc.at[pl.ds(s * RPT, RPT)])

    @pl.when(s == NS - 1)
    def _():
      pltpu.sync_copy(z_hbm.at[pl.ds(0, TAIL_Z)], acc.at[pl.ds(NS * RPT, TAIL_Z)])

    plsc.subcore_barrier()

    def g_start(j, buf, sem):
      pltpu.async_copy(h_hbm.at[idx_v.at[j]], buf, sem)

    def g_wait(j, buf, sem):
      pltpu.make_async_copy(h_hbm.at[idx_v.at[j]], buf, sem).wait()

    def s_start(j, buf, sem):
      pltpu.async_copy(buf, acc.at[idx_v.at[IBK + j]], sem, add=True)

    def s_wait(j, buf, sem):
      pltpu.make_async_copy(buf, acc.at[idx_v.at[IBK + j]], sem).wait()

    def process_block(nstreams):
      g_start(0, rows0, gsem0)
      for p in range(nstreams // 2 - 1):
        j = 2 * p
        g_wait(j, rows0, gsem0)
        s_start(j, rows0, ssem0)
        if j >= 1:
          s_wait(j - 1, rows1, ssem1)
        g_start(j + 1, rows1, gsem1)
        g_wait(j + 1, rows1, gsem1)
        s_start(j + 1, rows1, ssem1)
        s_wait(j, rows0, ssem0)
        g_start(j + 2, rows0, gsem0)
      g_wait(nstreams - 2, rows0, gsem0)
      s_start(nstreams - 2, rows0, ssem0)
      s_wait(nstreams - 3, rows1, ssem1)
      g_start(nstreams - 1, rows1, gsem1)
      g_wait(nstreams - 1, rows1, gsem1)
      s_start(nstreams - 1, rows1, ssem1)
      s_wait(nstreams - 2, rows0, ssem0)
      s_wait(nstreams - 1, rows1, ssem1)

    @pl.loop(0, NBLK)
    def _(i):
      pltpu.sync_copy(idx_hbm.at[c, s, i], idx_v)
      process_block(IBK)

    pltpu.sync_copy(idx_hbm.at[c, s, NBLK], idx_v)
    process_block(TBK)

    plsc.subcore_barrier()
    pltpu.sync_copy(acc.at[pl.ds(s * RPT, RPT)],
                    out_hbm.at[pl.ds(c * N + s * RPT, RPT)])

    @pl.when(s == NS - 1)
    def _():
      pltpu.sync_copy(acc.at[pl.ds(NS * RPT, TAIL_O)],
                      out_hbm.at[pl.ds(c * N + NS * RPT, TAIL_O)])

  return k(h2n, idx_all, zrows)


def _prep_edges(ei_u2i, ei_i2u):
  nblk_t = NBLK + 1
  pad_s = EPT - EPT_REAL
  pad_b = (nblk_t * IBK - SPT) * B

  def layout(flat, fill):
    per_tile = flat.reshape(NS, EPT_REAL)
    per_tile = jnp.concatenate(
        [per_tile, jnp.broadcast_to(fill, (NS, pad_s + pad_b)).astype(jnp.int32)],
        axis=1)
    return per_tile.reshape(NS, nblk_t, IBK, B)

  pad_dst = N + jnp.arange(pad_s + pad_b, dtype=jnp.int32) % (NPAD - N)
  blocks = []
  for rel, ei in enumerate((ei_u2i, ei_i2u)):
    src = layout(ei[0].astype(jnp.int32) + rel * N, rel * N)
    dst = layout(ei[1].astype(jnp.int32), pad_dst)
    blocks.append(jnp.concatenate([src, dst], axis=2))
  return jnp.stack(blocks)


def kernel(x_user, x_item, edge_index_u2i, edge_index_i2u,
           W0_u2i, b0_u2i, W0_i2u, b0_i2u,
           W1_u2i, b1_u2i, W1_i2u, b1_i2u):
  idx_all = _prep_edges(edge_index_u2i, edge_index_i2u)
  zrows = jnp.zeros((RPT, D), jnp.float32)

  x2n = jnp.concatenate([x_user, x_item], axis=0)
  h0 = _tc_layer(x2n, W0_u2i, b0_u2i, W0_i2u, b0_i2u, swap=False)
  agg0 = _sc_agg(h0, idx_all, zrows)
  h1 = _tc_layer(agg0, W1_u2i, b1_u2i, W1_i2u, b1_i2u, swap=True)
  agg1 = _sc_agg(h1, idx_all, zrows)
  return agg1[N:], agg1[:N]


# speedup vs baseline: 6.4089x; 1.0158x over previous
import functools

import jax
import jax.numpy as jnp
from jax import lax
from jax.experimental import pallas as pl
from jax.experimental.pallas import tpu as pltpu
from jax.experimental.pallas import tpu_sc as plsc

N = 10000
E = 320000
D = 128
NC = 2
NS = 16
B = 128
SPT = 158
IBK = 16
NBLK = SPT // IBK
TBK = SPT - NBLK * IBK
EPT = SPT * B
EPT_REAL = E // NS
E_PAD = EPT * NS
NPAD = N + 16
RPT = 624
TAIL_O = N - RPT * NS
TAIL_Z = NPAD - RPT * NS


def _tc_layer(x2n, Wa, ba, Wb, bb, swap):
  def body(x_ref, wa_ref, ba_ref, wb_ref, bb_ref, o_ref):
    if swap:
      xa = x_ref[N:, :]
      xb = x_ref[:N, :]
    else:
      xa = x_ref[:N, :]
      xb = x_ref[N:, :]
    ha = jnp.dot(xa, wa_ref[...], preferred_element_type=jnp.float32)
    o_ref[:N, :] = jnp.maximum(ha + ba_ref[...], 0.0)
    hb = jnp.dot(xb, wb_ref[...], preferred_element_type=jnp.float32)
    o_ref[N:, :] = jnp.maximum(hb + bb_ref[...], 0.0)

  return pl.pallas_call(
      body,
      out_shape=jax.ShapeDtypeStruct((2 * N, D), jnp.float32),
  )(x2n, Wa, ba.reshape(1, D), Wb, bb.reshape(1, D))


def _sc_agg(h2n, idx_all, zrows):
  mesh = plsc.VectorSubcoreMesh(core_axis_name="c", subcore_axis_name="s")

  @functools.partial(
      pl.kernel,
      mesh=mesh,
      out_type=jax.ShapeDtypeStruct((2 * N, D), jnp.float32),
      scratch_types=[
          pltpu.VMEM((2 * IBK, B), jnp.int32),
          pltpu.VMEM((2 * IBK, B), jnp.int32),
          pltpu.VMEM((B, D), jnp.float32),
          pltpu.VMEM((B, D), jnp.float32),
          pltpu.VMEM_SHARED((NPAD, D), jnp.float32),
          pltpu.SemaphoreType.DMA,
          pltpu.SemaphoreType.DMA,
          pltpu.SemaphoreType.DMA,
          pltpu.SemaphoreType.DMA,
          pltpu.SemaphoreType.DMA,
          pltpu.SemaphoreType.DMA,
      ],
  )
  def k(h_hbm, idx_hbm, z_hbm, out_hbm, idx0, idx1,
        rows0, rows1, acc, gsem0, gsem1, ssem0, ssem1, isem0, isem1):
    c = lax.axis_index("c")
    s = lax.axis_index("s")

    pltpu.sync_copy(z_hbm, acc.at[pl.ds(s * RPT, RPT)])

    @pl.when(s == NS - 1)
    def _():
      pltpu.sync_copy(z_hbm.at[pl.ds(0, TAIL_Z)], acc.at[pl.ds(NS * RPT, TAIL_Z)])

    plsc.subcore_barrier()

    def g_start(idx_v, j, buf, sem):
      pltpu.async_copy(h_hbm.at[idx_v.at[j]], buf, sem)

    def g_wait(idx_v, j, buf, sem):
      pltpu.make_async_copy(h_hbm.at[idx_v.at[j]], buf, sem).wait()

    def s_start(idx_v, j, buf, sem):
      pltpu.async_copy(buf, acc.at[idx_v.at[IBK + j]], sem, add=True)

    def s_wait(idx_v, j, buf, sem):
      pltpu.make_async_copy(buf, acc.at[idx_v.at[IBK + j]], sem).wait()

    def i_start(i, idx_v, sem):
      pltpu.async_copy(idx_hbm.at[c, s, i], idx_v, sem)

    def i_wait(i, idx_v, sem):
      pltpu.make_async_copy(idx_hbm.at[c, s, i], idx_v, sem).wait()

    def process_block(idx_v, nstreams):
      g_start(idx_v, 0, rows0, gsem0)
      for p in range(nstreams // 2 - 1):
        j = 2 * p
        g_wait(idx_v, j, rows0, gsem0)
        s_start(idx_v, j, rows0, ssem0)
        if j >= 1:
          s_wait(idx_v, j - 1, rows1, ssem1)
        g_start(idx_v, j + 1, rows1, gsem1)
        g_wait(idx_v, j + 1, rows1, gsem1)
        s_start(idx_v, j + 1, rows1, ssem1)
        s_wait(idx_v, j, rows0, ssem0)
        g_start(idx_v, j + 2, rows0, gsem0)
      g_wait(idx_v, nstreams - 2, rows0, gsem0)
      s_start(idx_v, nstreams - 2, rows0, ssem0)
      s_wait(idx_v, nstreams - 3, rows1, ssem1)
      g_start(idx_v, nstreams - 1, rows1, gsem1)
      g_wait(idx_v, nstreams - 1, rows1, gsem1)
      s_start(idx_v, nstreams - 1, rows1, ssem1)
      s_wait(idx_v, nstreams - 2, rows0, ssem0)
      s_wait(idx_v, nstreams - 1, rows1, ssem1)

    i_start(0, idx0, isem0)

    @pl.loop(0, NBLK // 2)
    def _(q):
      i = q * 2
      i_wait(i, idx0, isem0)
      i_start(i + 1, idx1, isem1)
      process_block(idx0, IBK)
      i_wait(i + 1, idx1, isem1)
      i_start(i + 2, idx0, isem0)
      process_block(idx1, IBK)

    i_wait(NBLK - 1, idx0, isem0)
    i_start(NBLK, idx1, isem1)
    process_block(idx0, IBK)
    i_wait(NBLK, idx1, isem1)
    process_block(idx1, TBK)

    plsc.subcore_barrier()
    pltpu.sync_copy(acc.at[pl.ds(s * RPT, RPT)],
                    out_hbm.at[pl.ds(c * N + s * RPT, RPT)])

    @pl.when(s == NS - 1)
    def _():
      pltpu.sync_copy(acc.at[pl.ds(NS * RPT, TAIL_O)],
                      out_hbm.at[pl.ds(c * N + NS * RPT, TAIL_O)])

  return k(h2n, idx_all, zrows)


def _prep_edges(ei_u2i, ei_i2u):
  nblk_t = NBLK + 1
  pad_s = EPT - EPT_REAL
  pad_b = (nblk_t * IBK - SPT) * B

  def layout(flat, fill):
    per_tile = flat.reshape(NS, EPT_REAL)
    per_tile = jnp.concatenate(
        [per_tile, jnp.broadcast_to(fill, (NS, pad_s + pad_b)).astype(jnp.int32)],
        axis=1)
    return per_tile.reshape(NS, nblk_t, IBK, B)

  pad_dst = N + jnp.arange(pad_s + pad_b, dtype=jnp.int32) % (NPAD - N)
  blocks = []
  for rel, ei in enumerate((ei_u2i, ei_i2u)):
    src = layout(ei[0].astype(jnp.int32) + rel * N, rel * N)
    dst = layout(ei[1].astype(jnp.int32), pad_dst)
    blocks.append(jnp.concatenate([src, dst], axis=2))
  return jnp.stack(blocks)


def kernel(x_user, x_item, edge_index_u2i, edge_index_i2u,
           W0_u2i, b0_u2i, W0_i2u, b0_i2u,
           W1_u2i, b1_u2i, W1_i2u, b1_i2u):
  idx_all = _prep_edges(edge_index_u2i, edge_index_i2u)
  zrows = jnp.zeros((RPT, D), jnp.float32)

  x2n = jnp.concatenate([x_user, x_item], axis=0)
  h0 = _tc_layer(x2n, W0_u2i, b0_u2i, W0_i2u, b0_i2u, swap=False)
  agg0 = _sc_agg(h0, idx_all, zrows)
  h1 = _tc_layer(agg0, W1_u2i, b1_u2i, W1_i2u, b1_i2u, swap=True)
  agg1 = _sc_agg(h1, idx_all, zrows)
  return agg1[N:], agg1[:N]


# speedup vs baseline: 6.4946x; 1.0134x over previous
import functools

import jax
import jax.numpy as jnp
from jax import lax
from jax.experimental import pallas as pl
from jax.experimental.pallas import tpu as pltpu
from jax.experimental.pallas import tpu_sc as plsc

N = 10000
E = 320000
D = 128
NC = 2
NS = 16
B = 128
SPT = 158
IBK = 16
NBLK = SPT // IBK
TBK = SPT - NBLK * IBK
EPT = SPT * B
EPT_REAL = E // NS
E_PAD = EPT * NS
NPAD = N + 16
RPT = 624
TAIL_O = N - RPT * NS
TAIL_Z = NPAD - RPT * NS


def _tc_layer0(xa, xb, Wa, ba, Wb, bb):
  def body(xa_ref, xb_ref, wa_ref, ba_ref, wb_ref, bb_ref, o_ref):
    ha = jnp.dot(xa_ref[...], wa_ref[...], preferred_element_type=jnp.float32)
    o_ref[:N, :] = jnp.maximum(ha + ba_ref[...], 0.0)
    hb = jnp.dot(xb_ref[...], wb_ref[...], preferred_element_type=jnp.float32)
    o_ref[N:, :] = jnp.maximum(hb + bb_ref[...], 0.0)

  return pl.pallas_call(
      body,
      out_shape=jax.ShapeDtypeStruct((2 * N, D), jnp.float32),
  )(xa, xb, Wa, ba.reshape(1, D), Wb, bb.reshape(1, D))


def _tc_layer1(x2n, Wa, ba, Wb, bb):
  def body(x_ref, wa_ref, ba_ref, wb_ref, bb_ref, o_ref):
    ha = jnp.dot(x_ref[N:, :], wa_ref[...], preferred_element_type=jnp.float32)
    o_ref[:N, :] = jnp.maximum(ha + ba_ref[...], 0.0)
    hb = jnp.dot(x_ref[:N, :], wb_ref[...], preferred_element_type=jnp.float32)
    o_ref[N:, :] = jnp.maximum(hb + bb_ref[...], 0.0)

  return pl.pallas_call(
      body,
      out_shape=jax.ShapeDtypeStruct((2 * N, D), jnp.float32),
  )(x2n, Wa, ba.reshape(1, D), Wb, bb.reshape(1, D))


def _sc_agg(h2n, idx_all, zrows):
  mesh = plsc.VectorSubcoreMesh(core_axis_name="c", subcore_axis_name="s")

  @functools.partial(
      pl.kernel,
      mesh=mesh,
      out_type=jax.ShapeDtypeStruct((2 * N, D), jnp.float32),
      scratch_types=[
          pltpu.VMEM((2 * IBK, B), jnp.int32),
          pltpu.VMEM((2 * IBK, B), jnp.int32),
          pltpu.VMEM((B, D), jnp.float32),
          pltpu.VMEM((B, D), jnp.float32),
          pltpu.VMEM_SHARED((NPAD, D), jnp.float32),
          pltpu.SemaphoreType.DMA,
          pltpu.SemaphoreType.DMA,
          pltpu.SemaphoreType.DMA,
          pltpu.SemaphoreType.DMA,
          pltpu.SemaphoreType.DMA,
          pltpu.SemaphoreType.DMA,
      ],
  )
  def k(h_hbm, idx_hbm, z_hbm, out_hbm, idx0, idx1,
        rows0, rows1, acc, gsem0, gsem1, ssem0, ssem1, isem0, isem1):
    c = lax.axis_index("c")
    s = lax.axis_index("s")

    pltpu.sync_copy(z_hbm, acc.at[pl.ds(s * RPT, RPT)])

    @pl.when(s == NS - 1)
    def _():
      pltpu.sync_copy(z_hbm.at[pl.ds(0, TAIL_Z)], acc.at[pl.ds(NS * RPT, TAIL_Z)])

    plsc.subcore_barrier()

    def g_start(idx_v, j, buf, sem):
      pltpu.async_copy(h_hbm.at[idx_v.at[j]], buf, sem)

    def g_wait(idx_v, j, buf, sem):
      pltpu.make_async_copy(h_hbm.at[idx_v.at[j]], buf, sem).wait()

    def s_start(idx_v, j, buf, sem):
      pltpu.async_copy(buf, acc.at[idx_v.at[IBK + j]], sem, add=True)

    def s_wait(idx_v, j, buf, sem):
      pltpu.make_async_copy(buf, acc.at[idx_v.at[IBK + j]], sem).wait()

    def i_start(i, idx_v, sem):
      pltpu.async_copy(idx_hbm.at[c, s, i], idx_v, sem)

    def i_wait(i, idx_v, sem):
      pltpu.make_async_copy(idx_hbm.at[c, s, i], idx_v, sem).wait()

    def process_block(idx_v, nstreams):
      g_start(idx_v, 0, rows0, gsem0)
      for p in range(nstreams // 2 - 1):
        j = 2 * p
        g_wait(idx_v, j, rows0, gsem0)
        s_start(idx_v, j, rows0, ssem0)
        if j >= 1:
          s_wait(idx_v, j - 1, rows1, ssem1)
        g_start(idx_v, j + 1, rows1, gsem1)
        g_wait(idx_v, j + 1, rows1, gsem1)
        s_start(idx_v, j + 1, rows1, ssem1)
        s_wait(idx_v, j, rows0, ssem0)
        g_start(idx_v, j + 2, rows0, gsem0)
      g_wait(idx_v, nstreams - 2, rows0, gsem0)
      s_start(idx_v, nstreams - 2, rows0, ssem0)
      s_wait(idx_v, nstreams - 3, rows1, ssem1)
      g_start(idx_v, nstreams - 1, rows1, gsem1)
      g_wait(idx_v, nstreams - 1, rows1, gsem1)
      s_start(idx_v, nstreams - 1, rows1, ssem1)
      s_wait(idx_v, nstreams - 2, rows0, ssem0)
      s_wait(idx_v, nstreams - 1, rows1, ssem1)

    i_start(0, idx0, isem0)

    @pl.loop(0, NBLK // 2)
    def _(q):
      i = q * 2
      i_wait(i, idx0, isem0)
      i_start(i + 1, idx1, isem1)
      process_block(idx0, IBK)
      i_wait(i + 1, idx1, isem1)
      i_start(i + 2, idx0, isem0)
      process_block(idx1, IBK)

    i_wait(NBLK - 1, idx0, isem0)
    i_start(NBLK, idx1, isem1)
    process_block(idx0, IBK)
    i_wait(NBLK, idx1, isem1)
    process_block(idx1, TBK)

    plsc.subcore_barrier()
    pltpu.sync_copy(acc.at[pl.ds(s * RPT, RPT)],
                    out_hbm.at[pl.ds(c * N + s * RPT, RPT)])

    @pl.when(s == NS - 1)
    def _():
      pltpu.sync_copy(acc.at[pl.ds(NS * RPT, TAIL_O)],
                      out_hbm.at[pl.ds(c * N + NS * RPT, TAIL_O)])

  return k(h2n, idx_all, zrows)


def _prep_edges(ei_u2i, ei_i2u):
  nblk_t = NBLK + 1
  pad_s = EPT - EPT_REAL
  pad_b = (nblk_t * IBK - SPT) * B

  def layout(flat, fill):
    per_tile = flat.reshape(NS, EPT_REAL)
    per_tile = jnp.concatenate(
        [per_tile, jnp.broadcast_to(fill, (NS, pad_s + pad_b)).astype(jnp.int32)],
        axis=1)
    return per_tile.reshape(NS, nblk_t, IBK, B)

  pad_dst = N + jnp.arange(pad_s + pad_b, dtype=jnp.int32) % (NPAD - N)
  blocks = []
  for rel, ei in enumerate((ei_u2i, ei_i2u)):
    src = layout(ei[0].astype(jnp.int32) + rel * N, rel * N)
    dst = layout(ei[1].astype(jnp.int32), pad_dst)
    blocks.append(jnp.concatenate([src, dst], axis=2))
  return jnp.stack(blocks)


def kernel(x_user, x_item, edge_index_u2i, edge_index_i2u,
           W0_u2i, b0_u2i, W0_i2u, b0_i2u,
           W1_u2i, b1_u2i, W1_i2u, b1_i2u):
  idx_all = _prep_edges(edge_index_u2i, edge_index_i2u)
  zrows = jnp.zeros((RPT, D), jnp.float32)

  h0 = _tc_layer0(x_user, x_item, W0_u2i, b0_u2i, W0_i2u, b0_i2u)
  agg0 = _sc_agg(h0, idx_all, zrows)
  h1 = _tc_layer1(agg0, W1_u2i, b1_u2i, W1_i2u, b1_i2u)
  agg1 = _sc_agg(h1, idx_all, zrows)
  return agg1[N:], agg1[:N]
